# baseline (device time: 266951 ns/iter reference)
import jax
import jax.numpy as jnp
from jax import lax
from jax.experimental import pallas as pl
from jax.experimental.pallas import tpu as pltpu

N_DEV = 4
SQ = 2048
SKV = 2048
HQ_TOTAL = 32
HG = 8
DH = 128
D_MODEL = 1024
D_HID = HG * DH
SCALE = 0.08838834764831843
BLK = 64
N_RES = 4
BLKS_PER_RES = (SQ // BLK) // N_RES
RROWS = BLKS_PER_RES * BLK


def kernel(x, Wq, K_ext, V_ext, Wo):
    x2 = x.reshape(SQ, D_MODEL)
    k2 = K_ext.reshape(SKV, HQ_TOTAL * DH)
    v2 = V_ext.reshape(SKV, HQ_TOTAL * DH)

    def body(x_ref, wq_ref, k_ref, v_ref, wo_ref, out_ref,
             comm_ref, kbuf, vbuf, send_sems, recv_sems, kv_sems):
        my = lax.axis_index("i")
        left = lax.rem(my + N_DEV - 1, N_DEV)
        right = lax.rem(my + 1, N_DEV)

        barrier = pltpu.get_barrier_semaphore()
        for nbr in (left, right):
            pl.semaphore_signal(barrier, inc=1, device_id=(nbr,),
                                device_id_type=pl.DeviceIdType.MESH)
        pl.semaphore_wait(barrier, 2)

        comm_ref[0, :D_MODEL, :] = wq_ref[...].astype(jnp.bfloat16)
        comm_ref[0, D_MODEL:, :] = wo_ref[...].astype(jnp.bfloat16)

        def compute(h, slot):
            g = lax.rem(my - h + N_DEV, N_DEV)
            col0 = g * D_HID
            kcopy = pltpu.make_async_copy(
                k_ref.at[:, pl.ds(col0, D_HID)], kbuf, kv_sems.at[0])
            vcopy = pltpu.make_async_copy(
                v_ref.at[:, pl.ds(col0, D_HID)], vbuf, kv_sems.at[1])
            kcopy.start()
            vcopy.start()

            wq_g = comm_ref[slot, :D_MODEL, :]
            wo_g = comm_ref[slot, D_MODEL:, :]
            qrs = []
            for r in range(N_RES):
                rows = [(r + N_RES * m) * BLK for m in range(BLKS_PER_RES)]
                xr = jnp.concatenate(
                    [x_ref[pl.ds(o, BLK), :] for o in rows], axis=0
                ).astype(jnp.bfloat16)
                qrs.append(jnp.dot(xr, wq_g,
                                   preferred_element_type=jnp.float32
                                   ).astype(jnp.bfloat16))
            kcopy.wait()
            vcopy.wait()

            for r in range(N_RES):
                rows = [(r + N_RES * m) * BLK for m in range(BLKS_PER_RES)]
                qr = qrs[r]
                kr = jnp.concatenate(
                    [kbuf[pl.ds(o, BLK), :] for o in rows], axis=0
                ).astype(jnp.bfloat16)
                vr = jnp.concatenate(
                    [vbuf[pl.ds(o, BLK), :] for o in rows], axis=0
                ).astype(jnp.bfloat16)
                ctx_cols = []
                for hh in range(HG):
                    c = hh * DH
                    qh = qr[:, c:c + DH]
                    kh = kr[:, c:c + DH]
                    s = lax.dot_general(
                        qh, kh, (((1,), (1,)), ((), ())),
                        preferred_element_type=jnp.float32) * SCALE
                    s = s - jnp.max(s, axis=1, keepdims=True)
                    w = jnp.exp(s)
                    w = (w / jnp.sum(w, axis=1, keepdims=True)
                         ).astype(jnp.bfloat16)
                    ctx_cols.append(
                        jnp.dot(w, vr[:, c:c + DH],
                                preferred_element_type=jnp.float32
                                ).astype(jnp.bfloat16))
                ctx_r = jnp.concatenate(ctx_cols, axis=1)
                part = jnp.dot(ctx_r, wo_g,
                               preferred_element_type=jnp.float32)
                for m in range(BLKS_PER_RES):
                    o = (r + N_RES * m) * BLK
                    blk = part[m * BLK:(m + 1) * BLK, :]
                    if h == 0:
                        out_ref[pl.ds(o, BLK), :] = blk
                    else:
                        out_ref[pl.ds(o, BLK), :] += blk

        for h in range(N_DEV):
            slot = h % 2
            if h < N_DEV - 1:
                rdma = pltpu.make_async_remote_copy(
                    src_ref=comm_ref.at[slot],
                    dst_ref=comm_ref.at[1 - slot],
                    send_sem=send_sems.at[slot],
                    recv_sem=recv_sems.at[1 - slot],
                    device_id=(right,),
                    device_id_type=pl.DeviceIdType.MESH,
                )
                rdma.start()
                compute(h, slot)
                rdma.wait()
            else:
                compute(h, slot)

    out = pl.pallas_call(
        body,
        out_shape=jax.ShapeDtypeStruct((SQ, D_MODEL), jnp.float32),
        in_specs=[
            pl.BlockSpec(memory_space=pltpu.VMEM),
            pl.BlockSpec(memory_space=pltpu.VMEM),
            pl.BlockSpec(memory_space=pl.ANY),
            pl.BlockSpec(memory_space=pl.ANY),
            pl.BlockSpec(memory_space=pltpu.VMEM),
        ],
        out_specs=pl.BlockSpec(memory_space=pltpu.VMEM),
        scratch_shapes=[
            pltpu.VMEM((2, 2 * D_MODEL, D_MODEL), jnp.bfloat16),
            pltpu.VMEM((SKV, D_HID), jnp.float32),
            pltpu.VMEM((SKV, D_HID), jnp.float32),
            pltpu.SemaphoreType.DMA((2,)),
            pltpu.SemaphoreType.DMA((2,)),
            pltpu.SemaphoreType.DMA((2,)),
        ],
        compiler_params=pltpu.CompilerParams(
            collective_id=0,
            vmem_limit_bytes=128 * 1024 * 1024,
        ),
    )(x2, Wq, k2, v2, Wo)
    return out.reshape(1, SQ, D_MODEL)


# device time: 184871 ns/iter; 1.4440x vs baseline; 1.4440x over previous
import jax
import jax.numpy as jnp
from jax import lax
from jax.experimental import pallas as pl
from jax.experimental.pallas import tpu as pltpu

N_DEV = 4
SQ = 2048
SKV = 2048
HQ_TOTAL = 32
HG = 8
DH = 128
D_MODEL = 1024
D_HID = HG * DH
SCALE = 0.08838834764831843
BLK = 64
N_RES = 4
BLKS_PER_RES = (SQ // BLK) // N_RES
RROWS = BLKS_PER_RES * BLK


def kernel(x, Wq, K_ext, V_ext, Wo):
    x2 = x.reshape(SQ, D_MODEL)

    def body(x_ref, wq_ref, k_ref, v_ref, wo_ref, out_ref,
             comm_ref, kbuf, vbuf, send_sems, recv_sems, kv_sems):
        my = lax.axis_index("i")
        left = lax.rem(my + N_DEV - 1, N_DEV)
        right = lax.rem(my + 1, N_DEV)

        barrier = pltpu.get_barrier_semaphore()
        for nbr in (left, right):
            pl.semaphore_signal(barrier, inc=1, device_id=(nbr,),
                                device_id_type=pl.DeviceIdType.MESH)
        pl.semaphore_wait(barrier, 2)

        comm_ref[0, :D_MODEL, :] = wq_ref[...].astype(jnp.bfloat16)
        comm_ref[0, D_MODEL:, :] = wo_ref[...].astype(jnp.bfloat16)

        def start_kv(h):
            g = lax.rem(my - h + N_DEV, N_DEV)
            copies = []
            for hh in range(HG):
                head = g * HG + hh
                copies.append(pltpu.make_async_copy(
                    k_ref.at[0, :, head, :], kbuf.at[hh], kv_sems.at[0]))
                copies.append(pltpu.make_async_copy(
                    v_ref.at[0, :, head, :], vbuf.at[hh], kv_sems.at[1]))
            for c in copies:
                c.start()
            return copies

        def compute(h, slot, kv_copies):
            wq_g = comm_ref[slot, :D_MODEL, :]
            wo_g = comm_ref[slot, D_MODEL:, :]
            qrs = []
            for r in range(N_RES):
                rows = [(r + N_RES * m) * BLK for m in range(BLKS_PER_RES)]
                xr = jnp.concatenate(
                    [x_ref[pl.ds(o, BLK), :] for o in rows], axis=0
                ).astype(jnp.bfloat16)
                qrs.append(jnp.dot(xr, wq_g,
                                   preferred_element_type=jnp.float32
                                   ).astype(jnp.bfloat16))
            for c in kv_copies:
                c.wait()

            for r in range(N_RES):
                rows = [(r + N_RES * m) * BLK for m in range(BLKS_PER_RES)]
                qr = qrs[r]
                ctx_cols = []
                for hh in range(HG):
                    c = hh * DH
                    qh = qr[:, c:c + DH]
                    kh = jnp.concatenate(
                        [kbuf[hh, pl.ds(o, BLK), :] for o in rows], axis=0
                    ).astype(jnp.bfloat16)
                    vh = jnp.concatenate(
                        [vbuf[hh, pl.ds(o, BLK), :] for o in rows], axis=0
                    ).astype(jnp.bfloat16)
                    s = lax.dot_general(
                        qh, kh, (((1,), (1,)), ((), ())),
                        preferred_element_type=jnp.float32) * SCALE
                    w = jnp.exp(s)
                    denom = jnp.sum(w, axis=1, keepdims=True)
                    ctx_h = jnp.dot(w.astype(jnp.bfloat16), vh,
                                    preferred_element_type=jnp.float32)
                    ctx_h = ctx_h * jax.lax.reciprocal(denom)
                    ctx_cols.append(ctx_h.astype(jnp.bfloat16))
                ctx_r = jnp.concatenate(ctx_cols, axis=1)
                part = jnp.dot(ctx_r, wo_g,
                               preferred_element_type=jnp.float32)
                for m in range(BLKS_PER_RES):
                    o = (r + N_RES * m) * BLK
                    blk = part[m * BLK:(m + 1) * BLK, :]
                    if h == 0:
                        out_ref[pl.ds(o, BLK), :] = blk
                    else:
                        out_ref[pl.ds(o, BLK), :] += blk

        for h in range(N_DEV):
            slot = h % 2
            kv_copies = start_kv(h)
            if h < N_DEV - 1:
                rdma = pltpu.make_async_remote_copy(
                    src_ref=comm_ref.at[slot],
                    dst_ref=comm_ref.at[1 - slot],
                    send_sem=send_sems.at[slot],
                    recv_sem=recv_sems.at[1 - slot],
                    device_id=(right,),
                    device_id_type=pl.DeviceIdType.MESH,
                )
                rdma.start()
                compute(h, slot, kv_copies)
                rdma.wait()
            else:
                compute(h, slot, kv_copies)

    out = pl.pallas_call(
        body,
        out_shape=jax.ShapeDtypeStruct((SQ, D_MODEL), jnp.float32),
        in_specs=[
            pl.BlockSpec(memory_space=pltpu.VMEM),
            pl.BlockSpec(memory_space=pltpu.VMEM),
            pl.BlockSpec(memory_space=pl.ANY),
            pl.BlockSpec(memory_space=pl.ANY),
            pl.BlockSpec(memory_space=pltpu.VMEM),
        ],
        out_specs=pl.BlockSpec(memory_space=pltpu.VMEM),
        scratch_shapes=[
            pltpu.VMEM((2, 2 * D_MODEL, D_MODEL), jnp.bfloat16),
            pltpu.VMEM((HG, SKV, DH), jnp.float32),
            pltpu.VMEM((HG, SKV, DH), jnp.float32),
            pltpu.SemaphoreType.DMA((2,)),
            pltpu.SemaphoreType.DMA((2,)),
            pltpu.SemaphoreType.DMA((2,)),
        ],
        compiler_params=pltpu.CompilerParams(
            collective_id=0,
            vmem_limit_bytes=128 * 1024 * 1024,
        ),
    )(x2, Wq, K_ext, V_ext, Wo)
    return out.reshape(1, SQ, D_MODEL)


# device time: 148399 ns/iter; 1.7989x vs baseline; 1.2458x over previous
import jax
import jax.numpy as jnp
from jax import lax
from jax.experimental import pallas as pl
from jax.experimental.pallas import tpu as pltpu

N_DEV = 4
SQ = 2048
SKV = 2048
HQ_TOTAL = 32
HG = 8
DH = 128
D_MODEL = 1024
D_HID = HG * DH
SCALE = 0.08838834764831843
BLK = 64
N_RES = 4
BLKS_PER_RES = (SQ // BLK) // N_RES
RROWS = BLKS_PER_RES * BLK


def kernel(x, Wq, K_ext, V_ext, Wo):
    x2 = x.reshape(SQ, D_MODEL).astype(jnp.bfloat16)
    wq = Wq.astype(jnp.bfloat16)
    wo = Wo.astype(jnp.bfloat16)

    def body(x_ref, wq_ref, k_ref, v_ref, wo_ref, out_ref,
             comm_ref, kbuf, vbuf, send_sems, recv_sems, kv_sems):
        my = lax.axis_index("i")
        left = lax.rem(my + N_DEV - 1, N_DEV)
        right = lax.rem(my + 1, N_DEV)

        barrier = pltpu.get_barrier_semaphore()
        for nbr in (left, right):
            pl.semaphore_signal(barrier, inc=1, device_id=(nbr,),
                                device_id_type=pl.DeviceIdType.MESH)
        pl.semaphore_wait(barrier, 2)

        comm_ref[0, :D_MODEL, :] = wq_ref[...]
        comm_ref[0, D_MODEL:, :] = wo_ref[...]

        def start_kv(g):
            copies = []
            for hh in range(HG):
                head = g * HG + hh
                copies.append(pltpu.make_async_copy(
                    k_ref.at[0, :, head, :], kbuf.at[hh], kv_sems.at[0]))
                copies.append(pltpu.make_async_copy(
                    v_ref.at[0, :, head, :], vbuf.at[hh], kv_sems.at[1]))
            for c in copies:
                c.start()
            return copies

        def compute(first, slot, kv_copies):
            wq_g = comm_ref[slot, :D_MODEL, :]
            wo_g = comm_ref[slot, D_MODEL:, :]
            qrs = []
            for r in range(N_RES):
                rows = [(r + N_RES * m) * BLK for m in range(BLKS_PER_RES)]
                xr = jnp.concatenate(
                    [x_ref[pl.ds(o, BLK), :] for o in rows], axis=0)
                qrs.append(jnp.dot(xr, wq_g,
                                   preferred_element_type=jnp.float32
                                   ).astype(jnp.bfloat16))
            for c in kv_copies:
                c.wait()

            for r in range(N_RES):
                rows = [(r + N_RES * m) * BLK for m in range(BLKS_PER_RES)]
                qr = qrs[r]
                ctx_cols = []
                for hh in range(HG):
                    c = hh * DH
                    qh = qr[:, c:c + DH]
                    kh = jnp.concatenate(
                        [kbuf[hh, pl.ds(o, BLK), :] for o in rows], axis=0
                    ).astype(jnp.bfloat16)
                    vh = jnp.concatenate(
                        [vbuf[hh, pl.ds(o, BLK), :] for o in rows], axis=0
                    ).astype(jnp.bfloat16)
                    s = lax.dot_general(
                        qh, kh, (((1,), (1,)), ((), ())),
                        preferred_element_type=jnp.float32) * SCALE
                    w = jnp.exp(s)
                    denom = jnp.sum(w, axis=1, keepdims=True)
                    ctx_h = jnp.dot(w.astype(jnp.bfloat16), vh,
                                    preferred_element_type=jnp.float32)
                    ctx_h = ctx_h * jax.lax.reciprocal(denom)
                    ctx_cols.append(ctx_h.astype(jnp.bfloat16))
                ctx_r = jnp.concatenate(ctx_cols, axis=1)
                part = jnp.dot(ctx_r, wo_g,
                               preferred_element_type=jnp.float32)
                for m in range(BLKS_PER_RES):
                    o = (r + N_RES * m) * BLK
                    blk = part[m * BLK:(m + 1) * BLK, :]
                    if first:
                        out_ref[pl.ds(o, BLK), :] = blk
                    else:
                        out_ref[pl.ds(o, BLK), :] += blk

        s1r = pltpu.make_async_remote_copy(
            src_ref=comm_ref.at[0], dst_ref=comm_ref.at[1],
            send_sem=send_sems.at[0], recv_sem=recv_sems.at[0],
            device_id=(right,), device_id_type=pl.DeviceIdType.MESH)
        s1l = pltpu.make_async_remote_copy(
            src_ref=comm_ref.at[0], dst_ref=comm_ref.at[2],
            send_sem=send_sems.at[1], recv_sem=recv_sems.at[1],
            device_id=(left,), device_id_type=pl.DeviceIdType.MESH)
        s1r.start()
        s1l.start()

        compute(True, 0, start_kv(my))

        s1r.wait_recv()
        s2r = pltpu.make_async_remote_copy(
            src_ref=comm_ref.at[1], dst_ref=comm_ref.at[3],
            send_sem=send_sems.at[2], recv_sem=recv_sems.at[2],
            device_id=(right,), device_id_type=pl.DeviceIdType.MESH)
        s2r.start()
        compute(False, 1, start_kv(left))

        s1l.wait_recv()
        compute(False, 2, start_kv(right))

        s2r.wait_recv()
        opp = lax.rem(my + 2, N_DEV)
        compute(False, 3, start_kv(opp))

        s1r.wait_send()
        s1l.wait_send()
        s2r.wait_send()

    out = pl.pallas_call(
        body,
        out_shape=jax.ShapeDtypeStruct((SQ, D_MODEL), jnp.float32),
        in_specs=[
            pl.BlockSpec(memory_space=pltpu.VMEM),
            pl.BlockSpec(memory_space=pltpu.VMEM),
            pl.BlockSpec(memory_space=pl.ANY),
            pl.BlockSpec(memory_space=pl.ANY),
            pl.BlockSpec(memory_space=pltpu.VMEM),
        ],
        out_specs=pl.BlockSpec(memory_space=pltpu.VMEM),
        scratch_shapes=[
            pltpu.VMEM((4, 2 * D_MODEL, D_MODEL), jnp.bfloat16),
            pltpu.VMEM((HG, SKV, DH), jnp.float32),
            pltpu.VMEM((HG, SKV, DH), jnp.float32),
            pltpu.SemaphoreType.DMA((3,)),
            pltpu.SemaphoreType.DMA((3,)),
            pltpu.SemaphoreType.DMA((2,)),
        ],
        compiler_params=pltpu.CompilerParams(
            collective_id=0,
            vmem_limit_bytes=128 * 1024 * 1024,
        ),
    )(x2, wq, K_ext, V_ext, wo)
    return out.reshape(1, SQ, D_MODEL)


# device time: 140050 ns/iter; 1.9061x vs baseline; 1.0596x over previous
import jax
import jax.numpy as jnp
from jax import lax
from jax.experimental import pallas as pl
from jax.experimental.pallas import tpu as pltpu

N_DEV = 4
SQ = 2048
SKV = 2048
HQ_TOTAL = 32
HG = 8
DH = 128
D_MODEL = 1024
D_HID = HG * DH
SCALE = 0.08838834764831843
BLK = 64
N_RES = 4
BLKS_PER_RES = (SQ // BLK) // N_RES
RROWS = BLKS_PER_RES * BLK


def kernel(x, Wq, K_ext, V_ext, Wo):
    x2 = x.reshape(SQ, D_MODEL).astype(jnp.bfloat16)
    wq = Wq.astype(jnp.bfloat16)
    wo = Wo.astype(jnp.bfloat16)
    k6 = K_ext.reshape(1, BLKS_PER_RES, N_RES, BLK, HQ_TOTAL, DH)
    v6 = V_ext.reshape(1, BLKS_PER_RES, N_RES, BLK, HQ_TOTAL, DH)

    def body(x_ref, wq_ref, k_ref, v_ref, wo_ref, out_ref,
             comm_ref, xp_ref, kbuf, vbuf, send_sems, recv_sems, kv_sems):
        my = lax.axis_index("i")
        left = lax.rem(my + N_DEV - 1, N_DEV)
        right = lax.rem(my + 1, N_DEV)

        barrier = pltpu.get_barrier_semaphore()
        for nbr in (left, right):
            pl.semaphore_signal(barrier, inc=1, device_id=(nbr,),
                                device_id_type=pl.DeviceIdType.MESH)
        pl.semaphore_wait(barrier, 2)

        comm_ref[0, :D_MODEL, :] = wq_ref[...]
        comm_ref[0, D_MODEL:, :] = wo_ref[...]

        for r in range(N_RES):
            for m in range(BLKS_PER_RES):
                xp_ref[pl.ds(r * RROWS + m * BLK, BLK), :] = (
                    x_ref[pl.ds((r + N_RES * m) * BLK, BLK), :])

        def start_kv(g):
            copies = []
            for hh in range(HG):
                head = g * HG + hh
                for r in range(N_RES):
                    copies.append(pltpu.make_async_copy(
                        k_ref.at[0, :, r, :, head, :], kbuf.at[hh, r],
                        kv_sems.at[0]))
                    copies.append(pltpu.make_async_copy(
                        v_ref.at[0, :, r, :, head, :], vbuf.at[hh, r],
                        kv_sems.at[1]))
            for c in copies:
                c.start()
            return copies

        def compute(first, slot, kv_copies):
            wq_g = comm_ref[slot, :D_MODEL, :]
            wo_g = comm_ref[slot, D_MODEL:, :]
            qrs = []
            for r in range(N_RES):
                xr = xp_ref[pl.ds(r * RROWS, RROWS), :]
                qrs.append(jnp.dot(xr, wq_g,
                                   preferred_element_type=jnp.float32
                                   ).astype(jnp.bfloat16))
            for c in kv_copies:
                c.wait()

            for r in range(N_RES):
                qr = qrs[r]
                ctx_cols = []
                for hh in range(HG):
                    c = hh * DH
                    qh = qr[:, c:c + DH]
                    kh = kbuf[hh, r].reshape(RROWS, DH).astype(jnp.bfloat16)
                    vh = vbuf[hh, r].reshape(RROWS, DH).astype(jnp.bfloat16)
                    s = lax.dot_general(
                        qh, kh, (((1,), (1,)), ((), ())),
                        preferred_element_type=jnp.float32) * SCALE
                    w = jnp.exp(s)
                    denom = jnp.sum(w, axis=1, keepdims=True)
                    ctx_h = jnp.dot(w.astype(jnp.bfloat16), vh,
                                    preferred_element_type=jnp.float32)
                    ctx_h = ctx_h * jax.lax.reciprocal(denom)
                    ctx_cols.append(ctx_h.astype(jnp.bfloat16))
                ctx_r = jnp.concatenate(ctx_cols, axis=1)
                part = jnp.dot(ctx_r, wo_g,
                               preferred_element_type=jnp.float32)
                for m in range(BLKS_PER_RES):
                    o = (r + N_RES * m) * BLK
                    blk = part[m * BLK:(m + 1) * BLK, :]
                    if first:
                        out_ref[pl.ds(o, BLK), :] = blk
                    else:
                        out_ref[pl.ds(o, BLK), :] += blk

        s1r = pltpu.make_async_remote_copy(
            src_ref=comm_ref.at[0], dst_ref=comm_ref.at[1],
            send_sem=send_sems.at[0], recv_sem=recv_sems.at[0],
            device_id=(right,), device_id_type=pl.DeviceIdType.MESH)
        s1l = pltpu.make_async_remote_copy(
            src_ref=comm_ref.at[0], dst_ref=comm_ref.at[2],
            send_sem=send_sems.at[1], recv_sem=recv_sems.at[1],
            device_id=(left,), device_id_type=pl.DeviceIdType.MESH)
        s1r.start()
        s1l.start()

        compute(True, 0, start_kv(my))

        s1r.wait_recv()
        s2r = pltpu.make_async_remote_copy(
            src_ref=comm_ref.at[1], dst_ref=comm_ref.at[3],
            send_sem=send_sems.at[2], recv_sem=recv_sems.at[2],
            device_id=(right,), device_id_type=pl.DeviceIdType.MESH)
        s2r.start()
        compute(False, 1, start_kv(left))

        s1l.wait_recv()
        compute(False, 2, start_kv(right))

        s2r.wait_recv()
        opp = lax.rem(my + 2, N_DEV)
        compute(False, 3, start_kv(opp))

        s1r.wait_send()
        s1l.wait_send()
        s2r.wait_send()

    out = pl.pallas_call(
        body,
        out_shape=jax.ShapeDtypeStruct((SQ, D_MODEL), jnp.float32),
        in_specs=[
            pl.BlockSpec(memory_space=pltpu.VMEM),
            pl.BlockSpec(memory_space=pltpu.VMEM),
            pl.BlockSpec(memory_space=pl.ANY),
            pl.BlockSpec(memory_space=pl.ANY),
            pl.BlockSpec(memory_space=pltpu.VMEM),
        ],
        out_specs=pl.BlockSpec(memory_space=pltpu.VMEM),
        scratch_shapes=[
            pltpu.VMEM((4, 2 * D_MODEL, D_MODEL), jnp.bfloat16),
            pltpu.VMEM((SQ, D_MODEL), jnp.bfloat16),
            pltpu.VMEM((HG, N_RES, BLKS_PER_RES, BLK, DH), jnp.float32),
            pltpu.VMEM((HG, N_RES, BLKS_PER_RES, BLK, DH), jnp.float32),
            pltpu.SemaphoreType.DMA((3,)),
            pltpu.SemaphoreType.DMA((3,)),
            pltpu.SemaphoreType.DMA((2,)),
        ],
        compiler_params=pltpu.CompilerParams(
            collective_id=0,
            vmem_limit_bytes=128 * 1024 * 1024,
        ),
    )(x2, wq, k6, v6, wo)
    return out.reshape(1, SQ, D_MODEL)


# device time: 123597 ns/iter; 2.1599x vs baseline; 1.1331x over previous
import jax
import jax.numpy as jnp
from jax import lax
from jax.experimental import pallas as pl
from jax.experimental.pallas import tpu as pltpu

N_DEV = 4
SQ = 2048
SKV = 2048
HQ_TOTAL = 32
HG = 8
DH = 128
D_MODEL = 1024
D_HID = HG * DH
SCALE = 0.08838834764831843
LOG2E = 1.4426950408889634
BLK = 64
N_RES = 4
BLKS_PER_RES = (SQ // BLK) // N_RES
RROWS = BLKS_PER_RES * BLK


def kernel(x, Wq, K_ext, V_ext, Wo):
    x2 = x.reshape(SQ, D_MODEL).astype(jnp.bfloat16)
    wq = Wq.astype(jnp.bfloat16)
    wo = Wo.astype(jnp.bfloat16)
    k6 = K_ext.reshape(1, BLKS_PER_RES, N_RES, BLK, HQ_TOTAL, DH)
    v6 = V_ext.reshape(1, BLKS_PER_RES, N_RES, BLK, HQ_TOTAL, DH)

    def body(x_ref, wq_ref, k_ref, v_ref, wo_ref, out_ref,
             comm_ref, xp_ref, kbuf, vbuf, send_sems, recv_sems, kv_sems):
        my = lax.axis_index("i")
        left = lax.rem(my + N_DEV - 1, N_DEV)
        right = lax.rem(my + 1, N_DEV)

        barrier = pltpu.get_barrier_semaphore()
        for nbr in (left, right):
            pl.semaphore_signal(barrier, inc=1, device_id=(nbr,),
                                device_id_type=pl.DeviceIdType.MESH)
        pl.semaphore_wait(barrier, 2)

        comm_ref[0, :D_MODEL, :] = wq_ref[...] * jnp.bfloat16(SCALE * LOG2E)
        comm_ref[0, D_MODEL:, :] = wo_ref[...]

        for r in range(N_RES):
            for m in range(BLKS_PER_RES):
                xp_ref[pl.ds(r * RROWS + m * BLK, BLK), :] = (
                    x_ref[pl.ds((r + N_RES * m) * BLK, BLK), :])

        def start_kv(g):
            copies = []
            for hh in range(HG):
                head = g * HG + hh
                for r in range(N_RES):
                    copies.append(pltpu.make_async_copy(
                        k_ref.at[0, :, r, :, head, :], kbuf.at[hh, r],
                        kv_sems.at[0]))
                    copies.append(pltpu.make_async_copy(
                        v_ref.at[0, :, r, :, head, :], vbuf.at[hh, r],
                        kv_sems.at[1]))
            for c in copies:
                c.start()
            return copies

        def compute(first, slot, kv_copies, wo_ready=None):
            wq_g = comm_ref[slot, :D_MODEL, :]
            qrs = []
            for r in range(N_RES):
                xr = xp_ref[pl.ds(r * RROWS, RROWS), :]
                qrs.append(jnp.dot(xr, wq_g,
                                   preferred_element_type=jnp.float32
                                   ).astype(jnp.bfloat16))
            for c in kv_copies:
                c.wait()

            ctxs = []
            for r in range(N_RES):
                qr = qrs[r]
                ctx_cols = []
                for hh in range(HG):
                    c = hh * DH
                    qh = qr[:, c:c + DH]
                    kh = kbuf[hh, r].reshape(RROWS, DH).astype(jnp.bfloat16)
                    vh = vbuf[hh, r].reshape(RROWS, DH).astype(jnp.bfloat16)
                    s = lax.dot_general(
                        qh, kh, (((1,), (1,)), ((), ())),
                        preferred_element_type=jnp.float32)
                    w = jnp.exp2(s)
                    denom = jnp.sum(w, axis=1, keepdims=True)
                    ctx_h = jnp.dot(w.astype(jnp.bfloat16), vh,
                                    preferred_element_type=jnp.float32)
                    ctx_h = ctx_h * jax.lax.reciprocal(denom)
                    ctx_cols.append(ctx_h.astype(jnp.bfloat16))
                ctxs.append(jnp.concatenate(ctx_cols, axis=1))

            if wo_ready is not None:
                wo_ready()
            wo_g = comm_ref[slot, D_MODEL:, :]
            for r in range(N_RES):
                part = jnp.dot(ctxs[r], wo_g,
                               preferred_element_type=jnp.float32)
                for m in range(BLKS_PER_RES):
                    o = (r + N_RES * m) * BLK
                    blk = part[m * BLK:(m + 1) * BLK, :]
                    if first:
                        out_ref[pl.ds(o, BLK), :] = blk
                    else:
                        out_ref[pl.ds(o, BLK), :] += blk

        def pair_send(dst_slot, target, sem0):
            wq_rdma = pltpu.make_async_remote_copy(
                src_ref=comm_ref.at[0, pl.ds(0, D_MODEL)],
                dst_ref=comm_ref.at[dst_slot, pl.ds(0, D_MODEL)],
                send_sem=send_sems.at[sem0], recv_sem=recv_sems.at[sem0],
                device_id=(target,), device_id_type=pl.DeviceIdType.MESH)
            wo_rdma = pltpu.make_async_remote_copy(
                src_ref=comm_ref.at[0, pl.ds(D_MODEL, D_MODEL)],
                dst_ref=comm_ref.at[dst_slot, pl.ds(D_MODEL, D_MODEL)],
                send_sem=send_sems.at[sem0 + 1], recv_sem=recv_sems.at[sem0 + 1],
                device_id=(target,), device_id_type=pl.DeviceIdType.MESH)
            return wq_rdma, wo_rdma

        s1r_wq, s1r_wo = pair_send(1, right, 0)
        s1l_wq, s1l_wo = pair_send(2, left, 2)
        s1r_wq.start()
        s1l_wq.start()
        s1r_wo.start()
        s1l_wo.start()

        compute(True, 0, start_kv(my))

        s1r_wq.wait_recv()
        s2r_wq = pltpu.make_async_remote_copy(
            src_ref=comm_ref.at[1, pl.ds(0, D_MODEL)],
            dst_ref=comm_ref.at[3, pl.ds(0, D_MODEL)],
            send_sem=send_sems.at[4], recv_sem=recv_sems.at[4],
            device_id=(right,), device_id_type=pl.DeviceIdType.MESH)
        s2r_wq.start()
        s2r_wo = pltpu.make_async_remote_copy(
            src_ref=comm_ref.at[1, pl.ds(D_MODEL, D_MODEL)],
            dst_ref=comm_ref.at[3, pl.ds(D_MODEL, D_MODEL)],
            send_sem=send_sems.at[5], recv_sem=recv_sems.at[5],
            device_id=(right,), device_id_type=pl.DeviceIdType.MESH)

        def wo1_ready():
            s1r_wo.wait_recv()
            s2r_wo.start()

        compute(False, 1, start_kv(left), wo1_ready)

        s1l_wq.wait_recv()
        compute(False, 2, start_kv(right), s1l_wo.wait_recv)

        s2r_wq.wait_recv()
        opp = lax.rem(my + 2, N_DEV)
        compute(False, 3, start_kv(opp), s2r_wo.wait_recv)

        for rdma in (s1r_wq, s1r_wo, s1l_wq, s1l_wo, s2r_wq, s2r_wo):
            rdma.wait_send()

    out = pl.pallas_call(
        body,
        out_shape=jax.ShapeDtypeStruct((SQ, D_MODEL), jnp.float32),
        in_specs=[
            pl.BlockSpec(memory_space=pltpu.VMEM),
            pl.BlockSpec(memory_space=pltpu.VMEM),
            pl.BlockSpec(memory_space=pl.ANY),
            pl.BlockSpec(memory_space=pl.ANY),
            pl.BlockSpec(memory_space=pltpu.VMEM),
        ],
        out_specs=pl.BlockSpec(memory_space=pltpu.VMEM),
        scratch_shapes=[
            pltpu.VMEM((4, 2 * D_MODEL, D_MODEL), jnp.bfloat16),
            pltpu.VMEM((SQ, D_MODEL), jnp.bfloat16),
            pltpu.VMEM((HG, N_RES, BLKS_PER_RES, BLK, DH), jnp.float32),
            pltpu.VMEM((HG, N_RES, BLKS_PER_RES, BLK, DH), jnp.float32),
            pltpu.SemaphoreType.DMA((6,)),
            pltpu.SemaphoreType.DMA((6,)),
            pltpu.SemaphoreType.DMA((2,)),
        ],
        compiler_params=pltpu.CompilerParams(
            collective_id=0,
            vmem_limit_bytes=128 * 1024 * 1024,
        ),
    )(x2, wq, k6, v6, wo)
    return out.reshape(1, SQ, D_MODEL)


# device time: 113318 ns/iter; 2.3558x vs baseline; 1.0907x over previous
import jax
import jax.numpy as jnp
from jax import lax
from jax.experimental import pallas as pl
from jax.experimental.pallas import tpu as pltpu

N_DEV = 4
SQ = 2048
SKV = 2048
HQ_TOTAL = 32
HG = 8
DH = 128
D_MODEL = 1024
D_HID = HG * DH
SCALE = 0.08838834764831843
LOG2E = 1.4426950408889634
BLK = 64
N_RES = 4
BLKS_PER_RES = (SQ // BLK) // N_RES
RROWS = BLKS_PER_RES * BLK


def kernel(x, Wq, K_ext, V_ext, Wo):
    x2 = x.reshape(SQ, D_MODEL)
    k6 = K_ext.reshape(1, BLKS_PER_RES, N_RES, BLK, HQ_TOTAL, DH)
    v6 = V_ext.reshape(1, BLKS_PER_RES, N_RES, BLK, HQ_TOTAL, DH)

    def body(x_ref, wq_ref, k_ref, v_ref, wo_ref, out_ref,
             comm_ref, xp_ref, xstage, kbuf, vbuf,
             send_sems, recv_sems, kv_sems, x_sems):
        my = lax.axis_index("i")
        left = lax.rem(my + N_DEV - 1, N_DEV)
        right = lax.rem(my + 1, N_DEV)

        barrier = pltpu.get_barrier_semaphore()
        for nbr in (left, right):
            pl.semaphore_signal(barrier, inc=1, device_id=(nbr,),
                                device_id_type=pl.DeviceIdType.MESH)
        pl.semaphore_wait(barrier, 2)

        comm_ref[0, :D_MODEL, :] = (
            wq_ref[...] * (SCALE * LOG2E)).astype(jnp.bfloat16)
        comm_ref[0, D_MODEL:, :] = wo_ref[...].astype(jnp.bfloat16)

        def permute_x():
            xcp = [None, None]

            def stage_x(r, buf):
                cs = [pltpu.make_async_copy(
                    x_ref.at[pl.ds((r + N_RES * m) * BLK, BLK)],
                    xstage.at[buf, pl.ds(m * BLK, BLK)],
                    x_sems.at[buf]) for m in range(BLKS_PER_RES)]
                for c in cs:
                    c.start()
                xcp[buf] = cs

            stage_x(0, 0)
            for r in range(N_RES):
                if r + 1 < N_RES:
                    stage_x(r + 1, (r + 1) % 2)
                for c in xcp[r % 2]:
                    c.wait()
                xp_ref[pl.ds(r * RROWS, RROWS), :] = (
                    xstage[r % 2].astype(jnp.bfloat16))

        def start_kv(g):
            copies = []
            for hh in range(HG):
                head = g * HG + hh
                for r in range(N_RES):
                    copies.append(pltpu.make_async_copy(
                        k_ref.at[0, :, r, :, head, :], kbuf.at[hh, r],
                        kv_sems.at[0]))
                    copies.append(pltpu.make_async_copy(
                        v_ref.at[0, :, r, :, head, :], vbuf.at[hh, r],
                        kv_sems.at[1]))
            for c in copies:
                c.start()
            return copies

        def compute(first, slot, kv_copies, wo_ready=None):
            wq_g = comm_ref[slot, :D_MODEL, :]
            qrs = []
            for r in range(N_RES):
                xr = xp_ref[pl.ds(r * RROWS, RROWS), :]
                qrs.append(jnp.dot(xr, wq_g,
                                   preferred_element_type=jnp.float32
                                   ).astype(jnp.bfloat16))
            for c in kv_copies:
                c.wait()

            ctxs = []
            for r in range(N_RES):
                qr = qrs[r]
                ctx_cols = []
                for hh in range(HG):
                    c = hh * DH
                    qh = qr[:, c:c + DH]
                    kh = kbuf[hh, r].reshape(RROWS, DH).astype(jnp.bfloat16)
                    vh = vbuf[hh, r].reshape(RROWS, DH).astype(jnp.bfloat16)
                    s = lax.dot_general(
                        qh, kh, (((1,), (1,)), ((), ())),
                        preferred_element_type=jnp.float32)
                    w = jnp.exp2(s)
                    denom = jnp.sum(w, axis=1, keepdims=True)
                    ctx_h = jnp.dot(w.astype(jnp.bfloat16), vh,
                                    preferred_element_type=jnp.float32)
                    ctx_h = ctx_h * jax.lax.reciprocal(denom)
                    ctx_cols.append(ctx_h.astype(jnp.bfloat16))
                ctxs.append(jnp.concatenate(ctx_cols, axis=1))

            if wo_ready is not None:
                wo_ready()
            wo_g = comm_ref[slot, D_MODEL:, :]
            for r in range(N_RES):
                part = jnp.dot(ctxs[r], wo_g,
                               preferred_element_type=jnp.float32)
                for m in range(BLKS_PER_RES):
                    o = (r + N_RES * m) * BLK
                    blk = part[m * BLK:(m + 1) * BLK, :]
                    if first:
                        out_ref[pl.ds(o, BLK), :] = blk
                    else:
                        out_ref[pl.ds(o, BLK), :] += blk

        def pair_send(dst_slot, target, sem0):
            wq_rdma = pltpu.make_async_remote_copy(
                src_ref=comm_ref.at[0, pl.ds(0, D_MODEL)],
                dst_ref=comm_ref.at[dst_slot, pl.ds(0, D_MODEL)],
                send_sem=send_sems.at[sem0], recv_sem=recv_sems.at[sem0],
                device_id=(target,), device_id_type=pl.DeviceIdType.MESH)
            wo_rdma = pltpu.make_async_remote_copy(
                src_ref=comm_ref.at[0, pl.ds(D_MODEL, D_MODEL)],
                dst_ref=comm_ref.at[dst_slot, pl.ds(D_MODEL, D_MODEL)],
                send_sem=send_sems.at[sem0 + 1], recv_sem=recv_sems.at[sem0 + 1],
                device_id=(target,), device_id_type=pl.DeviceIdType.MESH)
            return wq_rdma, wo_rdma

        s1r_wq, s1r_wo = pair_send(1, right, 0)
        s1l_wq, s1l_wo = pair_send(2, left, 2)
        s1r_wq.start()
        s1l_wq.start()
        s1r_wo.start()
        s1l_wo.start()

        own_kv = start_kv(my)
        permute_x()
        compute(True, 0, own_kv)

        s1r_wq.wait_recv()
        s2r_wq = pltpu.make_async_remote_copy(
            src_ref=comm_ref.at[1, pl.ds(0, D_MODEL)],
            dst_ref=comm_ref.at[3, pl.ds(0, D_MODEL)],
            send_sem=send_sems.at[4], recv_sem=recv_sems.at[4],
            device_id=(right,), device_id_type=pl.DeviceIdType.MESH)
        s2r_wq.start()
        s2r_wo = pltpu.make_async_remote_copy(
            src_ref=comm_ref.at[1, pl.ds(D_MODEL, D_MODEL)],
            dst_ref=comm_ref.at[3, pl.ds(D_MODEL, D_MODEL)],
            send_sem=send_sems.at[5], recv_sem=recv_sems.at[5],
            device_id=(right,), device_id_type=pl.DeviceIdType.MESH)

        def wo1_ready():
            s1r_wo.wait_recv()
            s2r_wo.start()

        compute(False, 1, start_kv(left), wo1_ready)

        s1l_wq.wait_recv()
        compute(False, 2, start_kv(right), s1l_wo.wait_recv)

        s2r_wq.wait_recv()
        opp = lax.rem(my + 2, N_DEV)
        compute(False, 3, start_kv(opp), s2r_wo.wait_recv)

        for rdma in (s1r_wq, s1r_wo, s1l_wq, s1l_wo, s2r_wq, s2r_wo):
            rdma.wait_send()

    out = pl.pallas_call(
        body,
        out_shape=jax.ShapeDtypeStruct((SQ, D_MODEL), jnp.float32),
        in_specs=[
            pl.BlockSpec(memory_space=pl.ANY),
            pl.BlockSpec(memory_space=pltpu.VMEM),
            pl.BlockSpec(memory_space=pl.ANY),
            pl.BlockSpec(memory_space=pl.ANY),
            pl.BlockSpec(memory_space=pltpu.VMEM),
        ],
        out_specs=pl.BlockSpec(memory_space=pltpu.VMEM),
        scratch_shapes=[
            pltpu.VMEM((4, 2 * D_MODEL, D_MODEL), jnp.bfloat16),
            pltpu.VMEM((SQ, D_MODEL), jnp.bfloat16),
            pltpu.VMEM((2, RROWS, D_MODEL), jnp.float32),
            pltpu.VMEM((HG, N_RES, BLKS_PER_RES, BLK, DH), jnp.float32),
            pltpu.VMEM((HG, N_RES, BLKS_PER_RES, BLK, DH), jnp.float32),
            pltpu.SemaphoreType.DMA((6,)),
            pltpu.SemaphoreType.DMA((6,)),
            pltpu.SemaphoreType.DMA((2,)),
            pltpu.SemaphoreType.DMA((2,)),
        ],
        compiler_params=pltpu.CompilerParams(
            collective_id=0,
            vmem_limit_bytes=128 * 1024 * 1024,
        ),
    )(x2, Wq, k6, v6, Wo)
    return out.reshape(1, SQ, D_MODEL)
